# speedup vs baseline: 6.3584x; 6.3584x over previous
import functools

import jax
import jax.numpy as jnp
from jax import lax
from jax.experimental import pallas as pl
from jax.experimental.pallas import tpu as pltpu
from jax.experimental.pallas import tpu_sc as plsc

EMBED_DIM = 128
CHUNK = 128
NUM_CORES = 2
NUM_SUBCORES = 16
NUM_WORKERS = NUM_CORES * NUM_SUBCORES


@functools.partial(jax.jit, static_argnums=(2,))
def _gather(table, idx2d, rows_per_worker):
    n_rows = idx2d.shape[0]

    mesh = plsc.VectorSubcoreMesh(core_axis_name="c", subcore_axis_name="s")

    @functools.partial(
        pl.kernel,
        mesh=mesh,
        out_type=jax.ShapeDtypeStruct((n_rows * CHUNK, EMBED_DIM), jnp.float32),
        scratch_types=[
            pltpu.VMEM((rows_per_worker, CHUNK), jnp.int32),
            pltpu.VMEM((CHUNK, EMBED_DIM), jnp.float32),
            pltpu.SemaphoreType.DMA,
        ],
    )
    def k(table_hbm, idx_hbm, out_hbm, idx_v, rows_v, sem):
        wid = lax.axis_index("s") * NUM_CORES + lax.axis_index("c")
        rbase = wid * rows_per_worker
        pltpu.sync_copy(idx_hbm.at[pl.ds(rbase, rows_per_worker)], idx_v)

        def chunk(j, carry):
            pltpu.async_copy(table_hbm.at[idx_v.at[j]], rows_v, sem).wait()
            pltpu.sync_copy(
                rows_v, out_hbm.at[pl.ds((rbase + j) * CHUNK, CHUNK)])
            return carry

        lax.fori_loop(0, rows_per_worker, chunk, 0)

    return k(table, idx2d)


def kernel(inputs, table):
    b, s = inputs.shape
    total = b * s
    n_rows = total // CHUNK
    idx2d = inputs.reshape(n_rows, CHUNK)
    out = _gather(table, idx2d, n_rows // NUM_WORKERS)
    return out.reshape(b, s, EMBED_DIM)


# speedup vs baseline: 9.1717x; 1.4425x over previous
import functools

import jax
import jax.numpy as jnp
from jax import lax
from jax.experimental import pallas as pl
from jax.experimental.pallas import tpu as pltpu
from jax.experimental.pallas import tpu_sc as plsc

EMBED_DIM = 128
CHUNK = 128
NBUF = 4
NUM_CORES = 2
NUM_SUBCORES = 16
NUM_WORKERS = NUM_CORES * NUM_SUBCORES


@functools.partial(jax.jit, static_argnums=(2,))
def _gather(table, idx2d, rows_per_worker):
    n_rows = idx2d.shape[0]
    n_groups = rows_per_worker // NBUF

    mesh = plsc.VectorSubcoreMesh(core_axis_name="c", subcore_axis_name="s")

    @functools.partial(
        pl.kernel,
        mesh=mesh,
        out_type=jax.ShapeDtypeStruct((n_rows * CHUNK, EMBED_DIM), jnp.float32),
        scratch_types=[
            pltpu.VMEM((rows_per_worker, CHUNK), jnp.int32),
            pltpu.VMEM((NBUF, CHUNK, EMBED_DIM), jnp.float32),
        ] + [pltpu.SemaphoreType.DMA] * (2 * NBUF),
    )
    def k(table_hbm, idx_hbm, out_hbm, idx_v, rows_v, *sems):
        gsem, osem = sems[:NBUF], sems[NBUF:]
        wid = lax.axis_index("s") * NUM_CORES + lax.axis_index("c")
        rbase = wid * rows_per_worker
        pltpu.sync_copy(idx_hbm.at[pl.ds(rbase, rows_per_worker)], idx_v)

        def fire_gather(j, b):
            pltpu.async_copy(table_hbm.at[idx_v.at[j]], rows_v.at[b], gsem[b])

        for b in range(NBUF):
            fire_gather(b, b)

        def group(g, carry):
            for b in range(NBUF):
                j = g * NBUF + b
                pltpu.make_async_copy(
                    table_hbm.at[idx_v.at[j]], rows_v.at[b], gsem[b]).wait()
                pltpu.async_copy(
                    rows_v.at[b],
                    out_hbm.at[pl.ds((rbase + j) * CHUNK, CHUNK)], osem[b])
            for b in range(NBUF):
                j = g * NBUF + b
                pltpu.make_async_copy(
                    rows_v.at[b],
                    out_hbm.at[pl.ds((rbase + j) * CHUNK, CHUNK)],
                    osem[b]).wait()

                @pl.when(g + 1 < n_groups)
                def _():
                    fire_gather((g + 1) * NBUF + b, b)

            return carry

        lax.fori_loop(0, n_groups, group, 0)

    return k(table, idx2d)


def kernel(inputs, table):
    b, s = inputs.shape
    total = b * s
    n_rows = total // CHUNK
    idx2d = inputs.reshape(n_rows, CHUNK)
    out = _gather(table, idx2d, n_rows // NUM_WORKERS)
    return out.reshape(b, s, EMBED_DIM)
